# baseline (speedup 1.0000x reference)
import functools

import jax
import jax.numpy as jnp
from jax import lax
from jax.experimental import pallas as pl
from jax.experimental.pallas import tpu as pltpu
from jax.experimental.pallas import tpu_sc as plsc


def _stage1_body(H, TAR, INP, VOCAB, JBS,
                 attn_ref, dec_ref, enc_row_ref, enc_col_ref, w_ref, b_ref,
                 lc_ref, gidx_ref):
  bi = pl.program_id(0)
  a = attn_ref[0]
  m = a[0:TAR, :]
  for h in range(1, H):
    m = m + a[h * TAR:(h + 1) * TAR, :]
  m = m * (1.0 / H)
  mmax = jnp.max(m, axis=-1, keepdims=True)
  e = jnp.exp(m - mmax)
  dist = e / jnp.sum(e, axis=-1, keepdims=True)
  x = jnp.dot(dec_ref[0], w_ref[...],
              preferred_element_type=jnp.float32) + b_ref[0, 0]
  pg = jax.nn.sigmoid(x)
  upd = (1.0 - pg) * dist
  enc_row = enc_row_ref[0]
  acc = jnp.zeros((TAR, INP), jnp.float32)
  for jb in range(INP // JBS):
    ej = enc_col_ref[0, pl.ds(jb * JBS, JBS), :]
    mjb = (ej == enc_row).astype(jnp.float32)
    acc = acc + jnp.dot(upd[:, jb * JBS:(jb + 1) * JBS], mjb,
                        preferred_element_type=jnp.float32)
  lc_ref[0] = jnp.log(acc)
  t_iota = lax.broadcasted_iota(jnp.int32, (TAR, INP), 0)
  gidx_ref[0] = (bi * TAR + t_iota) * VOCAB + enc_row


def _stage2_body(fin_ref, dec_ref, w_ref, b_ref, out_ref, lse_ref, lpg_ref):
  x = jnp.dot(dec_ref[...], w_ref[...],
              preferred_element_type=jnp.float32) + b_ref[0, 0]
  lpg = jax.nn.log_sigmoid(x)
  row = fin_ref[...]
  mmax = jnp.max(row, axis=-1, keepdims=True)
  lse = mmax + jnp.log(jnp.sum(jnp.exp(row - mmax), axis=-1, keepdims=True))
  out_ref[...] = row - lse + lpg
  lse_ref[...] = lse
  lpg_ref[...] = lpg


def _stage4_body(fing_ref, lse_ref, lpg_ref, lc_ref, fix_ref):
  g = fing_ref[...] - lse_ref[...] + lpg_ref[...]
  fix_ref[...] = jnp.logaddexp(g, lc_ref[...])


def kernel(dec_output, final_output, attention_weights, encoder_input,
           W, b, inp_shape, tar_shape, training):
  B, TAR, D = dec_output.shape
  VOCAB = final_output.shape[-1]
  H = attention_weights.shape[1]
  INP = encoder_input.shape[1]
  R = B * TAR
  N = R * VOCAB
  JBS = 512

  attn_r = attention_weights.reshape(B, H * TAR, INP)
  enc_row = encoder_input.reshape(B, 1, INP)
  enc_col = encoder_input.reshape(B, INP, 1)
  b2 = b.reshape(1, 1)
  dec2 = dec_output.reshape(R, D)
  fin2 = final_output.reshape(R, VOCAB)

  lc, gidx = pl.pallas_call(
      functools.partial(_stage1_body, H, TAR, INP, VOCAB, JBS),
      grid=(B,),
      in_specs=[
          pl.BlockSpec((1, H * TAR, INP), lambda i: (i, 0, 0)),
          pl.BlockSpec((1, TAR, D), lambda i: (i, 0, 0)),
          pl.BlockSpec((1, 1, INP), lambda i: (i, 0, 0)),
          pl.BlockSpec((1, INP, 1), lambda i: (i, 0, 0)),
          pl.BlockSpec((D, 1), lambda i: (0, 0)),
          pl.BlockSpec((1, 1), lambda i: (0, 0)),
      ],
      out_specs=[
          pl.BlockSpec((1, TAR, INP), lambda i: (i, 0, 0)),
          pl.BlockSpec((1, TAR, INP), lambda i: (i, 0, 0)),
      ],
      out_shape=[
          jax.ShapeDtypeStruct((B, TAR, INP), jnp.float32),
          jax.ShapeDtypeStruct((B, TAR, INP), jnp.int32),
      ],
  )(attn_r, dec_output, enc_row, enc_col, W, b2)

  RB = 8
  dense, lse, lpg = pl.pallas_call(
      _stage2_body,
      grid=(R // RB,),
      in_specs=[
          pl.BlockSpec((RB, VOCAB), lambda i: (i, 0)),
          pl.BlockSpec((RB, D), lambda i: (i, 0)),
          pl.BlockSpec((D, 1), lambda i: (0, 0)),
          pl.BlockSpec((1, 1), lambda i: (0, 0)),
      ],
      out_specs=[
          pl.BlockSpec((RB, VOCAB), lambda i: (i, 0)),
          pl.BlockSpec((RB, 1), lambda i: (i, 0)),
          pl.BlockSpec((RB, 1), lambda i: (i, 0)),
      ],
      out_shape=[
          jax.ShapeDtypeStruct((R, VOCAB), jnp.float32),
          jax.ShapeDtypeStruct((R, 1), jnp.float32),
          jax.ShapeDtypeStruct((R, 1), jnp.float32),
      ],
  )(fin2, dec2, W, b2)

  NC, NS = 2, 16
  NW = NC * NS
  K = R * INP
  CW = K // NW
  C = K // (NW * CW)
  mesh = plsc.VectorSubcoreMesh(core_axis_name="c", subcore_axis_name="s")
  gidx3 = gidx.reshape(NW, CW)
  fin_flat = final_output.reshape(N)

  @functools.partial(
      pl.kernel,
      out_type=jax.ShapeDtypeStruct((NW, CW), jnp.float32),
      mesh=mesh,
      scratch_types=[
          pltpu.VMEM((CW,), jnp.int32),
          pltpu.VMEM((CW,), jnp.float32),
          pltpu.SemaphoreType.DMA,
      ],
  )
  def sc_gather(fin_hbm, idx_hbm, out_hbm, idx_v, val_v, sem):
    w = lax.axis_index("s") * NC + lax.axis_index("c")
    pltpu.sync_copy(idx_hbm.at[w], idx_v)
    pltpu.async_copy(fin_hbm.at[idx_v], val_v, sem).wait()
    pltpu.sync_copy(val_v, out_hbm.at[w])

  fin_g = sc_gather(fin_flat, gidx3)

  fix = pl.pallas_call(
      _stage4_body,
      grid=(1,),
      in_specs=[
          pl.BlockSpec((R, INP), lambda i: (0, 0)),
          pl.BlockSpec((R, 1), lambda i: (0, 0)),
          pl.BlockSpec((R, 1), lambda i: (0, 0)),
          pl.BlockSpec((R, INP), lambda i: (0, 0)),
      ],
      out_specs=pl.BlockSpec((R, INP), lambda i: (0, 0)),
      out_shape=jax.ShapeDtypeStruct((R, INP), jnp.float32),
  )(fin_g.reshape(R, INP), lse, lpg, lc.reshape(R, INP))

  @functools.partial(
      pl.kernel,
      out_type=(),
      mesh=mesh,
      scratch_types=[
          pltpu.VMEM((CW,), jnp.int32),
          pltpu.VMEM((CW,), jnp.float32),
          pltpu.SemaphoreType.DMA,
      ],
  )
  def sc_scatter(idx_hbm, fix_hbm, dense_ref, idx_v, val_v, sem):
    w = lax.axis_index("s") * NC + lax.axis_index("c")
    pltpu.sync_copy(idx_hbm.at[w], idx_v)
    pltpu.sync_copy(fix_hbm.at[w], val_v)
    pltpu.async_copy(val_v, dense_ref.at[idx_v], sem).wait()

  dense_ref = jax.new_ref(dense.reshape(N))
  sc_scatter(gidx3, fix.reshape(NW, CW), dense_ref)
  return dense_ref[...].reshape(B, TAR, VOCAB)


# speedup vs baseline: 4.0304x; 4.0304x over previous
import functools

import jax
import jax.numpy as jnp
from jax import lax
from jax.experimental import pallas as pl
from jax.experimental.pallas import tpu as pltpu
from jax.experimental.pallas import tpu_sc as plsc


def _stage1_body(H, TAR, attn_ref, dec_ref, w_ref, b_ref, upd_ref):
  a = attn_ref[0]
  m = a[0:TAR, :]
  for h in range(1, H):
    m = m + a[h * TAR:(h + 1) * TAR, :]
  m = m * (1.0 / H)
  mmax = jnp.max(m, axis=-1, keepdims=True)
  e = jnp.exp(m - mmax)
  dist = e / jnp.sum(e, axis=-1, keepdims=True)
  x = jnp.dot(dec_ref[0], w_ref[...],
              preferred_element_type=jnp.float32) + b_ref[0, 0]
  pg = jax.nn.sigmoid(x)
  upd_ref[0] = (1.0 - pg) * dist


def _stage2_body(fin_ref, dec_ref, w_ref, b_ref, out_ref):
  x = jnp.dot(dec_ref[...], w_ref[...],
              preferred_element_type=jnp.float32) + b_ref[0, 0]
  pg = jax.nn.sigmoid(x)
  row = fin_ref[...]
  mmax = jnp.max(row, axis=-1, keepdims=True)
  e = jnp.exp(row - mmax)
  s = jnp.sum(e, axis=-1, keepdims=True)
  out_ref[...] = e * (pg / s)


def _stage3_body(RB, probs_ref, out_ref):
  out_ref[0] = jnp.log(probs_ref[...])


def kernel(dec_output, final_output, attention_weights, encoder_input,
           W, b, inp_shape, tar_shape, training):
  B, TAR, D = dec_output.shape
  VOCAB = final_output.shape[-1]
  H = attention_weights.shape[1]
  INP = encoder_input.shape[1]
  R = B * TAR

  attn_r = attention_weights.reshape(B, H * TAR, INP)
  b2 = b.reshape(1, 1)
  dec2 = dec_output.reshape(R, D)
  fin2 = final_output.reshape(R, VOCAB)

  upd = pl.pallas_call(
      functools.partial(_stage1_body, H, TAR),
      grid=(B,),
      in_specs=[
          pl.BlockSpec((1, H * TAR, INP), lambda i: (i, 0, 0)),
          pl.BlockSpec((1, TAR, D), lambda i: (i, 0, 0)),
          pl.BlockSpec((D, 1), lambda i: (0, 0)),
          pl.BlockSpec((1, 1), lambda i: (0, 0)),
      ],
      out_specs=pl.BlockSpec((1, TAR, INP), lambda i: (i, 0, 0)),
      out_shape=jax.ShapeDtypeStruct((B, TAR, INP), jnp.float32),
  )(attn_r, dec_output, W, b2)

  RB = 8
  probs = pl.pallas_call(
      _stage2_body,
      grid=(R // RB,),
      in_specs=[
          pl.BlockSpec((RB, VOCAB), lambda i: (i, 0)),
          pl.BlockSpec((RB, D), lambda i: (i, 0)),
          pl.BlockSpec((D, 1), lambda i: (0, 0)),
          pl.BlockSpec((1, 1), lambda i: (0, 0)),
      ],
      out_specs=pl.BlockSpec((RB, VOCAB), lambda i: (i, 0)),
      out_shape=jax.ShapeDtypeStruct((R, VOCAB), jnp.float32),
  )(fin2, dec2, W, b2)

  NC, NS = 2, 16
  NW = NC * NS
  RPW = R // NW
  L = 16
  mesh = plsc.VectorSubcoreMesh(core_axis_name="c", subcore_axis_name="s")
  upd2 = upd.reshape(R, INP)

  @functools.partial(
      pl.kernel,
      out_type=(),
      mesh=mesh,
      compiler_params=pltpu.CompilerParams(needs_layout_passes=False),
      scratch_types=[
          pltpu.VMEM((VOCAB,), jnp.float32),
          pltpu.VMEM((INP,), jnp.int32),
          pltpu.VMEM((INP,), jnp.float32),
          pltpu.SemaphoreType.DMA,
      ],
  )
  def sc_scatter_add(enc_hbm, upd_hbm, probs_ref, row_v, idx_v, val_v, sem):
    w = lax.axis_index("s") * NC + lax.axis_index("c")
    bi = (w * RPW) // TAR
    pltpu.sync_copy(enc_hbm.at[bi], idx_v)
    for j in range(RPW):
      r = w * RPW + j
      pltpu.sync_copy(probs_ref.at[r], row_v)
      pltpu.sync_copy(upd_hbm.at[r], val_v)

      @pl.loop(0, INP // L)
      def _(k):
        iv = idx_v[pl.ds(k * L, L)]
        vv = val_v[pl.ds(k * L, L)]
        plsc.addupdate_scatter(row_v, [iv], vv)

      pltpu.sync_copy(row_v, probs_ref.at[r])

  probs_ref = jax.new_ref(probs)
  sc_scatter_add(encoder_input, upd2, probs_ref)
  combined = probs_ref[...]

  out = pl.pallas_call(
      functools.partial(_stage3_body, RB),
      grid=(R // RB,),
      in_specs=[pl.BlockSpec((RB, VOCAB), lambda i: (i, 0))],
      out_specs=pl.BlockSpec((1, RB, VOCAB),
                             lambda i: (i // (TAR // RB), i % (TAR // RB), 0)),
      out_shape=jax.ShapeDtypeStruct((B, TAR, VOCAB), jnp.float32),
  )(combined)
  return out


# speedup vs baseline: 4.4120x; 1.0947x over previous
import functools

import jax
import jax.numpy as jnp
from jax import lax
from jax.experimental import pallas as pl
from jax.experimental.pallas import tpu as pltpu
from jax.experimental.pallas import tpu_sc as plsc

_NCHUNK = 4


def _stage1_body(H, TAR, attn_ref, dec_ref, w_ref, b_ref, upd_ref):
  a = attn_ref[0]
  m = a[0:TAR, :]
  for h in range(1, H):
    m = m + a[h * TAR:(h + 1) * TAR, :]
  m = m * (1.0 / H)
  mmax = jnp.max(m, axis=-1, keepdims=True)
  e = jnp.exp(m - mmax)
  dist = e / jnp.sum(e, axis=-1, keepdims=True)
  x = jnp.dot(dec_ref[0], w_ref[...],
              preferred_element_type=jnp.float32) + b_ref[0, 0]
  pg = jax.nn.sigmoid(x)
  upd_ref[0] = (1.0 - pg) * dist


def _stage2_body(fin_ref, dec_ref, w_ref, b_ref, out_ref):
  x = jnp.dot(dec_ref[...], w_ref[...],
              preferred_element_type=jnp.float32) + b_ref[0, 0]
  pg = jax.nn.sigmoid(x)
  row = fin_ref[...]
  mmax = jnp.max(row, axis=-1, keepdims=True)
  e = jnp.exp(row - mmax)
  s = jnp.sum(e, axis=-1, keepdims=True)
  out_ref[...] = e * (pg / s)


def _stage4_body(probs_ref, prev_ref, out_ref):
  del prev_ref
  out_ref[0] = jnp.log(probs_ref[...])


def kernel(dec_output, final_output, attention_weights, encoder_input,
           W, b, inp_shape, tar_shape, training):
  B, TAR, D = dec_output.shape
  VOCAB = final_output.shape[-1]
  H = attention_weights.shape[1]
  INP = encoder_input.shape[1]
  R = B * TAR
  RC = R // _NCHUNK

  attn_r = attention_weights.reshape(B, H * TAR, INP)
  b2 = b.reshape(1, 1)
  dec2 = dec_output.reshape(R, D)
  fin2 = final_output.reshape(R, VOCAB)

  upd = pl.pallas_call(
      functools.partial(_stage1_body, H, TAR),
      grid=(B,),
      in_specs=[
          pl.BlockSpec((1, H * TAR, INP), lambda i: (i, 0, 0)),
          pl.BlockSpec((1, TAR, D), lambda i: (i, 0, 0)),
          pl.BlockSpec((D, 1), lambda i: (0, 0)),
          pl.BlockSpec((1, 1), lambda i: (0, 0)),
      ],
      out_specs=pl.BlockSpec((1, TAR, INP), lambda i: (i, 0, 0)),
      out_shape=jax.ShapeDtypeStruct((B, TAR, INP), jnp.float32),
  )(attn_r, dec_output, W, b2)
  upd2 = upd.reshape(R, INP)

  NC, NS = 2, 16
  NW = NC * NS
  RPW = RC // NW
  L = 16
  mesh = plsc.VectorSubcoreMesh(core_axis_name="c", subcore_axis_name="s")
  RB = 8

  def make_sc_chunk(c):
    @functools.partial(
        pl.kernel,
        out_type=(),
        mesh=mesh,
        compiler_params=pltpu.CompilerParams(needs_layout_passes=False),
        scratch_types=[
            pltpu.VMEM((VOCAB,), jnp.float32),
            pltpu.VMEM((INP,), jnp.int32),
            pltpu.VMEM((INP,), jnp.float32),
            pltpu.SemaphoreType.DMA,
        ],
    )
    def sc_scatter_add(enc_hbm, upd_hbm, probs_ref, row_v, idx_v, val_v, sem):
      w = lax.axis_index("s") * NC + lax.axis_index("c")
      bi = (c * RC + w * RPW) // TAR
      pltpu.sync_copy(enc_hbm.at[bi], idx_v)
      for j in range(RPW):
        r = w * RPW + j
        pltpu.sync_copy(probs_ref.at[r], row_v)
        pltpu.sync_copy(upd_hbm.at[c * RC + r], val_v)

        @pl.loop(0, INP // L)
        def _(k):
          iv = idx_v[pl.ds(k * L, L)]
          vv = val_v[pl.ds(k * L, L)]
          plsc.addupdate_scatter(row_v, [iv], vv)

        pltpu.sync_copy(row_v, probs_ref.at[r])

    return sc_scatter_add

  fixed = dict(dec=dec2, W=W, b2=b2)
  combined_chunks = []
  for c in range(_NCHUNK):
    probs_c = pl.pallas_call(
        _stage2_body,
        grid=(RC // RB,),
        in_specs=[
            pl.BlockSpec((RB, VOCAB), lambda j, c=c: (c * (RC // RB) + j, 0)),
            pl.BlockSpec((RB, D), lambda j, c=c: (c * (RC // RB) + j, 0)),
            pl.BlockSpec((D, 1), lambda j: (0, 0)),
            pl.BlockSpec((1, 1), lambda j: (0, 0)),
        ],
        out_specs=pl.BlockSpec((RB, VOCAB), lambda j: (j, 0)),
        out_shape=jax.ShapeDtypeStruct((RC, VOCAB), jnp.float32),
    )(fin2, fixed["dec"], fixed["W"], fixed["b2"])
    ref_c = jax.new_ref(probs_c)
    make_sc_chunk(c)(encoder_input, upd2, ref_c)
    combined_chunks.append(ref_c[...])

  BPC = RC // TAR
  TB = TAR // RB
  out = None
  for c in range(_NCHUNK):
    if out is None:
      args = (combined_chunks[c], jnp.zeros((1, 1), jnp.float32))
      prev_spec = pl.BlockSpec((1, 1), lambda j: (0, 0))
      aliases = {}
    else:
      args = (combined_chunks[c], out)
      prev_spec = pl.BlockSpec(memory_space=pl.ANY)
      aliases = {1: 0}
    out = pl.pallas_call(
        _stage4_body,
        grid=(RC // RB,),
        in_specs=[pl.BlockSpec((RB, VOCAB), lambda j: (j, 0)), prev_spec],
        out_specs=pl.BlockSpec(
            (1, RB, VOCAB),
            lambda j, c=c: (c * BPC + j // TB, j % TB, 0)),
        out_shape=jax.ShapeDtypeStruct((B, TAR, VOCAB), jnp.float32),
        input_output_aliases=aliases,
    )(*args)
  return out
